# baseline (speedup 1.0000x reference)
import functools

import jax
import jax.numpy as jnp
from jax import lax
from jax.experimental import pallas as pl
from jax.experimental.pallas import tpu as pltpu
from jax.experimental.pallas import tpu_sc as plsc

D_MODEL = 768
SEQ = 512
NC = 2
NS = 16
NW = NC * NS
P = SEQ // NW
LANES = 16
DJ = D_MODEL // LANES


def _pe_table():
    even_i = jnp.arange(0, D_MODEL, 2, dtype=jnp.float32)
    denominator = jnp.power(10000.0, even_i / D_MODEL)
    position = jnp.arange(SEQ, dtype=jnp.float32).reshape(SEQ, 1)
    even = jnp.sin(position / denominator)
    odd = jnp.cos(position / denominator)
    return jnp.stack([even, odd], axis=2).reshape(SEQ, D_MODEL)


def _make_sc_embed(B, V):
    mesh = plsc.VectorSubcoreMesh(core_axis_name="c", subcore_axis_name="s")

    NPAIR = P // 2

    @functools.partial(
        pl.kernel,
        mesh=mesh,
        out_type=jax.ShapeDtypeStruct((B, SEQ, D_MODEL), jnp.float32),
        scratch_types=[
            pltpu.VMEM((P, B), jnp.int32),
            pltpu.VMEM((P, D_MODEL), jnp.float32),
            pltpu.VMEM((B, D_MODEL), jnp.float32),
            pltpu.VMEM((B, D_MODEL), jnp.float32),
            pltpu.SemaphoreType.DMA,
            pltpu.SemaphoreType.DMA,
            pltpu.SemaphoreType.DMA,
            pltpu.SemaphoreType.DMA,
        ],
    )
    def sc_embed(xt_hbm, pe_hbm, table_hbm, out_hbm,
                 idx_v, pe_v, buf0, buf1, g0, g1, s0, s1):
        w = lax.axis_index("s") * NC + lax.axis_index("c")
        wp = w * P
        pltpu.sync_copy(xt_hbm.at[pl.ds(wp, P), :], idx_v)
        pltpu.sync_copy(pe_hbm.at[pl.ds(wp, P), :], pe_v)

        def gather(p, buf, sem):
            pltpu.make_async_copy(table_hbm.at[idx_v.at[p]], buf, sem).start()

        def wait_gather(buf, sem):
            pltpu.make_async_copy(table_hbm.at[idx_v.at[0]], buf, sem).wait()

        def store(p, buf, sem):
            pltpu.make_async_copy(buf, out_hbm.at[:, wp + p, :], sem).start()

        def wait_store(buf, sem):
            pltpu.make_async_copy(buf, out_hbm.at[:, wp, :], sem).wait()

        def add_pe(p, buf):
            for j in range(DJ):
                dsj = pl.ds(LANES * j, LANES)
                pe_vec = pe_v[p, dsj]

                @plsc.parallel_loop(0, B, step=1, unroll=4)
                def _(b):
                    buf[b, dsj] = buf[b, dsj] + pe_vec

        gather(0, buf0, g0)
        gather(1, buf1, g1)

        def pair(i, carry):
            p0 = 2 * i
            p1 = p0 + 1
            wait_gather(buf0, g0)
            add_pe(p0, buf0)
            store(p0, buf0, s0)
            wait_gather(buf1, g1)
            add_pe(p1, buf1)
            store(p1, buf1, s1)

            @pl.when(i < NPAIR - 1)
            def _():
                wait_store(buf0, s0)
                gather(p0 + 2, buf0, g0)
                wait_store(buf1, s1)
                gather(p1 + 2, buf1, g1)

            return carry

        lax.fori_loop(0, NPAIR, pair, 0)
        wait_store(buf0, s0)
        wait_store(buf1, s1)

    return sc_embed


def kernel(x, table):
    B = x.shape[0]
    V = table.shape[0]
    pe = _pe_table()
    xt = jnp.transpose(x.astype(jnp.int32))
    return _make_sc_embed(B, V)(xt, pe, table)


# speedup vs baseline: 1.1085x; 1.1085x over previous
import functools

import jax
import jax.numpy as jnp
from jax import lax
from jax.experimental import pallas as pl
from jax.experimental.pallas import tpu as pltpu
from jax.experimental.pallas import tpu_sc as plsc

D_MODEL = 768
SEQ = 512
NC = 2
NS = 16
NW = NC * NS
P = SEQ // NW
LANES = 16
DJ = D_MODEL // LANES


def _pe_table():
    even_i = jnp.arange(0, D_MODEL, 2, dtype=jnp.float32)
    denominator = jnp.power(10000.0, even_i / D_MODEL)
    position = jnp.arange(SEQ, dtype=jnp.float32).reshape(SEQ, 1)
    even = jnp.sin(position / denominator)
    odd = jnp.cos(position / denominator)
    return jnp.stack([even, odd], axis=2).reshape(SEQ, D_MODEL)


def _make_sc_embed(B, V):
    mesh = plsc.VectorSubcoreMesh(core_axis_name="c", subcore_axis_name="s")

    NPAIR = P // 2

    @functools.partial(
        pl.kernel,
        mesh=mesh,
        out_type=jax.ShapeDtypeStruct((B, SEQ, D_MODEL), jnp.float32),
        scratch_types=[
            pltpu.VMEM((P, B), jnp.int32),
            pltpu.VMEM((P, D_MODEL), jnp.float32),
            pltpu.VMEM((B, D_MODEL), jnp.float32),
            pltpu.VMEM((B, D_MODEL), jnp.float32),
            pltpu.SemaphoreType.DMA,
            pltpu.SemaphoreType.DMA,
            pltpu.SemaphoreType.DMA,
            pltpu.SemaphoreType.DMA,
        ],
    )
    def sc_embed(xt_hbm, pe_hbm, table_hbm, out_hbm,
                 idx_v, pe_v, buf0, buf1, g0, g1, s0, s1):
        w = lax.axis_index("s") * NC + lax.axis_index("c")
        wp = w * P
        pltpu.sync_copy(xt_hbm.at[pl.ds(wp, P), :], idx_v)
        pltpu.sync_copy(pe_hbm.at[pl.ds(wp, P), :], pe_v)

        def gather(p, buf, sem):
            pltpu.make_async_copy(table_hbm.at[idx_v.at[p]], buf, sem).start()

        def wait_gather(buf, sem):
            pltpu.make_async_copy(table_hbm.at[idx_v.at[0]], buf, sem).wait()

        def store(p, buf, sem):
            pltpu.make_async_copy(buf, out_hbm.at[:, wp + p, :], sem).start()

        def wait_store(buf, sem):
            pltpu.make_async_copy(buf, out_hbm.at[:, wp, :], sem).wait()

        def add_pe(p, buf):
            for j in range(DJ):
                dsj = pl.ds(LANES * j, LANES)
                pe_vec = pe_v[p, dsj]

                @plsc.parallel_loop(0, B, step=1, unroll=8)
                def _(b):
                    buf[b, dsj] = buf[b, dsj] + pe_vec

        gather(0, buf0, g0)
        gather(1, buf1, g1)

        def pair(i, carry):
            p0 = 2 * i
            p1 = p0 + 1
            wait_gather(buf0, g0)
            add_pe(p0, buf0)
            store(p0, buf0, s0)
            wait_gather(buf1, g1)
            add_pe(p1, buf1)
            store(p1, buf1, s1)

            @pl.when(i < NPAIR - 1)
            def _():
                wait_store(buf0, s0)
                gather(p0 + 2, buf0, g0)
                wait_store(buf1, s1)
                gather(p1 + 2, buf1, g1)

            return carry

        lax.fori_loop(0, NPAIR, pair, 0)
        wait_store(buf0, s0)
        wait_store(buf1, s1)

    return sc_embed


def kernel(x, table):
    B = x.shape[0]
    V = table.shape[0]
    pe = _pe_table()
    xt = jnp.transpose(x.astype(jnp.int32))
    return _make_sc_embed(B, V)(xt, pe, table)
